# baseline (speedup 1.0000x reference)
import functools

import jax
import jax.numpy as jnp
from jax import lax
from jax.experimental import pallas as pl
from jax.experimental.pallas import tpu as pltpu
from jax.experimental.pallas import tpu_sc as plsc

N_NODES = 10000
N_EDGES = 320000
D_IN = 128
D_DEG = 16
D_OUT = 128

NC = 2
NS = 16
NW = NC * NS
EDGES_PER_WORKER = N_EDGES // NW
CHUNK = 40
N_CHUNKS = EDGES_PER_WORKER // CHUNK
N_ACC = 10240
ROWS_PER_TILE = N_ACC // NS
NROW = 6
NIDX = 8

_mesh = plsc.VectorSubcoreMesh(
    core_axis_name="c", subcore_axis_name="s", num_cores=NC, num_subcores=NS
)


@functools.partial(
    pl.kernel,
    out_type=(
        jax.ShapeDtypeStruct((NC * N_ACC, D_IN), jnp.float32),
        jax.ShapeDtypeStruct((NC * N_ACC, D_DEG), jnp.float32),
    ),
    mesh=_mesh,
    scratch_types=[
        pltpu.VMEM((NIDX, CHUNK), jnp.int32),
        pltpu.VMEM((NIDX, CHUNK), jnp.int32),
        pltpu.VMEM((NROW, CHUNK, D_IN), jnp.float32),
        pltpu.VMEM((CHUNK, D_DEG), jnp.float32),
        pltpu.VMEM((CHUNK, D_DEG), jnp.float32),
        pltpu.VMEM_SHARED((N_ACC, D_IN), jnp.float32),
        pltpu.VMEM_SHARED((N_ACC, D_DEG), jnp.float32),
        pltpu.SemaphoreType.DMA,
        pltpu.SemaphoreType.DMA,
        pltpu.SemaphoreType.DMA,
        pltpu.SemaphoreType.DMA,
        pltpu.SemaphoreType.DMA,
    ],
    compiler_params=pltpu.CompilerParams(use_tc_tiling_on_sc=False),
)
def _sc_aggregate(x_hbm, edge_hbm, feat_hbm, deg_hbm, ridx_v, cidx_v, rows_v,
                  ones_v, zd_v, feat_sh, deg_sh, sem_g, sem_s, sem_d, sem_ir,
                  sem_ic):
    c = lax.axis_index("c")
    s = lax.axis_index("s")
    wid = s * NC + c
    ebase = wid * EDGES_PER_WORKER

    def _load_idx(i, b):
        off = ebase + i * CHUNK
        pltpu.async_copy(edge_hbm.at[0, pl.ds(off, CHUNK)], ridx_v.at[b],
                         sem_ir)
        pltpu.async_copy(edge_hbm.at[1, pl.ds(off, CHUNK)], cidx_v.at[b],
                         sem_ic)

    def _wait_cidx():
        pltpu.make_async_copy(edge_hbm.at[1, pl.ds(0, CHUNK)], cidx_v.at[0],
                              sem_ic).wait()

    def _wait_ridx():
        pltpu.make_async_copy(edge_hbm.at[0, pl.ds(0, CHUNK)], ridx_v.at[0],
                              sem_ir).wait()

    def _gather(i, b):
        pltpu.async_copy(x_hbm.at[cidx_v.at[lax.rem(i, NIDX)]], rows_v.at[b],
                         sem_g)

    def _wait_gather():
        pltpu.make_async_copy(x_hbm.at[cidx_v.at[0]], rows_v.at[0],
                              sem_g).wait()

    def _wait_scatters():
        pltpu.make_async_copy(
            rows_v.at[0], feat_sh.at[ridx_v.at[0]], sem_s).wait()
        pltpu.make_async_copy(
            ones_v, deg_sh.at[ridx_v.at[0]], sem_d).wait()

    _load_idx(0, 0)
    _load_idx(1, 1)

    def _zero_row(i, carry):
        for j in range(D_IN // 16):
            rows_v[0, i, pl.ds(j * 16, 16)] = jnp.zeros((16,), jnp.float32)
        ones_v[i, :] = jnp.ones((D_DEG,), jnp.float32)
        zd_v[i, :] = jnp.zeros((D_DEG,), jnp.float32)
        return carry

    lax.fori_loop(0, CHUNK, _zero_row, 0)
    row_base = pl.multiple_of(s * ROWS_PER_TILE, 8)
    for k in range(ROWS_PER_TILE // CHUNK):
        pltpu.async_copy(rows_v.at[0],
                         feat_sh.at[pl.ds(row_base + k * CHUNK, CHUNK)], sem_s)
        pltpu.async_copy(zd_v, deg_sh.at[pl.ds(row_base + k * CHUNK, CHUNK)],
                         sem_d)
    for k in range(ROWS_PER_TILE // CHUNK):
        pltpu.make_async_copy(
            rows_v.at[0], feat_sh.at[pl.ds(row_base, CHUNK)], sem_s).wait()
        pltpu.make_async_copy(
            zd_v, deg_sh.at[pl.ds(row_base, CHUNK)], sem_d).wait()

    _wait_cidx()
    _gather(0, 0)
    plsc.subcore_barrier()

    def _edge_chunk(i, carry):
        @pl.when(i >= NROW - 1)
        def _():
            _wait_scatters()

        @pl.when(i + 1 < N_CHUNKS)
        def _():
            _wait_cidx()
            _gather(i + 1, lax.rem(i + 1, NROW))

        @pl.when(i + 2 < N_CHUNKS)
        def _():
            _load_idx(i + 2, lax.rem(i + 2, NIDX))

        _wait_gather()
        _wait_ridx()
        p = lax.rem(i, NROW)
        b = lax.rem(i, NIDX)
        pltpu.async_copy(rows_v.at[p], feat_sh.at[ridx_v.at[b]], sem_s,
                         add=True)
        pltpu.async_copy(ones_v, deg_sh.at[ridx_v.at[b]], sem_d, add=True)
        return carry

    lax.fori_loop(0, N_CHUNKS, _edge_chunk, 0)
    for _ in range(NROW - 1):
        _wait_scatters()
    plsc.subcore_barrier()

    out_base = pl.multiple_of(c * N_ACC + row_base, 8)
    pltpu.async_copy(feat_sh.at[pl.ds(row_base, ROWS_PER_TILE)],
                     feat_hbm.at[pl.ds(out_base, ROWS_PER_TILE)], sem_s)
    pltpu.async_copy(deg_sh.at[pl.ds(row_base, ROWS_PER_TILE)],
                     deg_hbm.at[pl.ds(out_base, ROWS_PER_TILE)], sem_d)
    pltpu.make_async_copy(
        feat_sh.at[pl.ds(row_base, ROWS_PER_TILE)],
        feat_hbm.at[pl.ds(out_base, ROWS_PER_TILE)], sem_s).wait()
    pltpu.make_async_copy(
        deg_sh.at[pl.ds(row_base, ROWS_PER_TILE)],
        deg_hbm.at[pl.ds(out_base, ROWS_PER_TILE)], sem_d).wait()


TC_BLK = 2560


def _tc_finalize(fa_ref, fb_ref, da_ref, db_ref, wlin_ref, blin_ref, wsq_ref,
                 out_ref):
    f = fa_ref[...] + fb_ref[...]
    d = da_ref[:, 0:1] + db_ref[:, 0:1]
    norm = f / jnp.maximum(d, 1.0)
    h = lax.dot_general(norm, wlin_ref[...], (((1,), (1,)), ((), ())),
                        preferred_element_type=jnp.float32)
    h = h + blin_ref[...]
    out_ref[...] = jnp.dot(h, wsq_ref[...], preferred_element_type=jnp.float32)


def kernel(x, edge_index, W_lin, b_lin, weight):
    feat, deg = _sc_aggregate(x, edge_index.astype(jnp.int32))
    nblk_half = N_ACC // TC_BLK
    out = pl.pallas_call(
        _tc_finalize,
        grid=((N_NODES + TC_BLK - 1) // TC_BLK,),
        in_specs=[
            pl.BlockSpec((TC_BLK, D_IN), lambda i: (i, 0)),
            pl.BlockSpec((TC_BLK, D_IN), lambda i: (i + nblk_half, 0)),
            pl.BlockSpec((TC_BLK, D_DEG), lambda i: (i, 0)),
            pl.BlockSpec((TC_BLK, D_DEG), lambda i: (i + nblk_half, 0)),
            pl.BlockSpec((D_OUT, D_IN), lambda i: (0, 0)),
            pl.BlockSpec((1, D_OUT), lambda i: (0, 0)),
            pl.BlockSpec((D_OUT, D_OUT), lambda i: (0, 0)),
        ],
        out_specs=pl.BlockSpec((TC_BLK, D_OUT), lambda i: (i, 0)),
        out_shape=jax.ShapeDtypeStruct((N_NODES, D_OUT), jnp.float32),
    )(feat, feat, deg, deg, W_lin, b_lin.reshape(1, D_OUT), weight)
    return out


# speedup vs baseline: 1.2704x; 1.2704x over previous
import functools

import jax
import jax.numpy as jnp
from jax import lax
from jax.experimental import pallas as pl
from jax.experimental.pallas import tpu as pltpu
from jax.experimental.pallas import tpu_sc as plsc

N_NODES = 10000
N_EDGES = 320000
D_IN = 128
D_DEG = 16
D_OUT = 128

NC = 2
NS = 16
NW = NC * NS
EDGES_PER_WORKER = N_EDGES // NW
CHUNK = 80
N_CHUNKS = EDGES_PER_WORKER // CHUNK
N_ACC = 10240
ROWS_PER_TILE = N_ACC // NS
NROW = 3
NIDX = 4

_mesh = plsc.VectorSubcoreMesh(
    core_axis_name="c", subcore_axis_name="s", num_cores=NC, num_subcores=NS
)


@functools.partial(
    pl.kernel,
    out_type=(
        jax.ShapeDtypeStruct((NC * N_ACC, D_IN), jnp.float32),
        jax.ShapeDtypeStruct((NC * N_ACC, D_DEG), jnp.float32),
    ),
    mesh=_mesh,
    scratch_types=[
        pltpu.VMEM((NIDX, CHUNK), jnp.int32),
        pltpu.VMEM((NIDX, CHUNK), jnp.int32),
        pltpu.VMEM((NROW, CHUNK, D_IN), jnp.float32),
        pltpu.VMEM((CHUNK, D_DEG), jnp.float32),
        pltpu.VMEM((CHUNK, D_DEG), jnp.float32),
        pltpu.VMEM_SHARED((N_ACC, D_IN), jnp.float32),
        pltpu.VMEM_SHARED((N_ACC, D_DEG), jnp.float32),
        pltpu.SemaphoreType.DMA,
        pltpu.SemaphoreType.DMA,
        pltpu.SemaphoreType.DMA,
        pltpu.SemaphoreType.DMA,
        pltpu.SemaphoreType.DMA,
    ],
    compiler_params=pltpu.CompilerParams(use_tc_tiling_on_sc=False),
)
def _sc_aggregate(x_hbm, edge_hbm, feat_hbm, deg_hbm, ridx_v, cidx_v, rows_v,
                  ones_v, zd_v, feat_sh, deg_sh, sem_g, sem_s, sem_d, sem_ir,
                  sem_ic):
    c = lax.axis_index("c")
    s = lax.axis_index("s")
    wid = s * NC + c
    ebase = wid * EDGES_PER_WORKER

    def _load_idx(i, b):
        off = ebase + i * CHUNK
        pltpu.async_copy(edge_hbm.at[0, pl.ds(off, CHUNK)], ridx_v.at[b],
                         sem_ir)
        pltpu.async_copy(edge_hbm.at[1, pl.ds(off, CHUNK)], cidx_v.at[b],
                         sem_ic)

    def _wait_cidx():
        pltpu.make_async_copy(edge_hbm.at[1, pl.ds(0, CHUNK)], cidx_v.at[0],
                              sem_ic).wait()

    def _wait_ridx():
        pltpu.make_async_copy(edge_hbm.at[0, pl.ds(0, CHUNK)], ridx_v.at[0],
                              sem_ir).wait()

    def _gather(i, b):
        pltpu.async_copy(x_hbm.at[cidx_v.at[lax.rem(i, NIDX)]], rows_v.at[b],
                         sem_g)

    def _wait_gather():
        pltpu.make_async_copy(x_hbm.at[cidx_v.at[0]], rows_v.at[0],
                              sem_g).wait()

    def _wait_scatters():
        pltpu.make_async_copy(
            rows_v.at[0], feat_sh.at[ridx_v.at[0]], sem_s).wait()
        pltpu.make_async_copy(
            ones_v, deg_sh.at[ridx_v.at[0]], sem_d).wait()

    _load_idx(0, 0)
    _load_idx(1, 1)

    def _zero_row(i, carry):
        for j in range(D_IN // 16):
            rows_v[0, i, pl.ds(j * 16, 16)] = jnp.zeros((16,), jnp.float32)
        ones_v[i, :] = jnp.ones((D_DEG,), jnp.float32)
        zd_v[i, :] = jnp.zeros((D_DEG,), jnp.float32)
        return carry

    lax.fori_loop(0, CHUNK, _zero_row, 0)
    row_base = pl.multiple_of(s * ROWS_PER_TILE, 8)
    for k in range(ROWS_PER_TILE // CHUNK):
        pltpu.async_copy(rows_v.at[0],
                         feat_sh.at[pl.ds(row_base + k * CHUNK, CHUNK)], sem_s)
        pltpu.async_copy(zd_v, deg_sh.at[pl.ds(row_base + k * CHUNK, CHUNK)],
                         sem_d)
    for k in range(ROWS_PER_TILE // CHUNK):
        pltpu.make_async_copy(
            rows_v.at[0], feat_sh.at[pl.ds(row_base, CHUNK)], sem_s).wait()
        pltpu.make_async_copy(
            zd_v, deg_sh.at[pl.ds(row_base, CHUNK)], sem_d).wait()

    _wait_cidx()
    _gather(0, 0)
    plsc.subcore_barrier()

    def _edge_chunk(i, carry):
        @pl.when(i >= 2)
        def _():
            _wait_scatters()

        @pl.when(i + 1 < N_CHUNKS)
        def _():
            _wait_cidx()
            _gather(i + 1, lax.rem(i + 1, NROW))

        @pl.when(i + 2 < N_CHUNKS)
        def _():
            _load_idx(i + 2, lax.rem(i + 2, NIDX))

        _wait_gather()
        _wait_ridx()
        p = lax.rem(i, NROW)
        b = lax.rem(i, NIDX)
        pltpu.async_copy(rows_v.at[p], feat_sh.at[ridx_v.at[b]], sem_s,
                         add=True)
        pltpu.async_copy(ones_v, deg_sh.at[ridx_v.at[b]], sem_d, add=True)
        return carry

    lax.fori_loop(0, N_CHUNKS, _edge_chunk, 0)
    _wait_scatters()
    _wait_scatters()
    plsc.subcore_barrier()

    out_base = pl.multiple_of(c * N_ACC + row_base, 8)
    pltpu.async_copy(feat_sh.at[pl.ds(row_base, ROWS_PER_TILE)],
                     feat_hbm.at[pl.ds(out_base, ROWS_PER_TILE)], sem_s)
    pltpu.async_copy(deg_sh.at[pl.ds(row_base, ROWS_PER_TILE)],
                     deg_hbm.at[pl.ds(out_base, ROWS_PER_TILE)], sem_d)
    pltpu.make_async_copy(
        feat_sh.at[pl.ds(row_base, ROWS_PER_TILE)],
        feat_hbm.at[pl.ds(out_base, ROWS_PER_TILE)], sem_s).wait()
    pltpu.make_async_copy(
        deg_sh.at[pl.ds(row_base, ROWS_PER_TILE)],
        deg_hbm.at[pl.ds(out_base, ROWS_PER_TILE)], sem_d).wait()


TC_BLK = 5120


def _tc_finalize(fa_ref, fb_ref, da_ref, db_ref, wlin_ref, blin_ref, wsq_ref,
                 out_ref):
    f = fa_ref[...] + fb_ref[...]
    d = da_ref[:, 0:1] + db_ref[:, 0:1]
    norm = f / jnp.maximum(d, 1.0)
    h = lax.dot_general(norm, wlin_ref[...], (((1,), (1,)), ((), ())),
                        preferred_element_type=jnp.float32)
    h = h + blin_ref[...]
    out_ref[...] = jnp.dot(h, wsq_ref[...], preferred_element_type=jnp.float32)


def kernel(x, edge_index, W_lin, b_lin, weight):
    feat, deg = _sc_aggregate(x, edge_index.astype(jnp.int32))
    nblk_half = N_ACC // TC_BLK
    out = pl.pallas_call(
        _tc_finalize,
        grid=((N_NODES + TC_BLK - 1) // TC_BLK,),
        in_specs=[
            pl.BlockSpec((TC_BLK, D_IN), lambda i: (i, 0)),
            pl.BlockSpec((TC_BLK, D_IN), lambda i: (i + nblk_half, 0)),
            pl.BlockSpec((TC_BLK, D_DEG), lambda i: (i, 0)),
            pl.BlockSpec((TC_BLK, D_DEG), lambda i: (i + nblk_half, 0)),
            pl.BlockSpec((D_OUT, D_IN), lambda i: (0, 0)),
            pl.BlockSpec((1, D_OUT), lambda i: (0, 0)),
            pl.BlockSpec((D_OUT, D_OUT), lambda i: (0, 0)),
        ],
        out_specs=pl.BlockSpec((TC_BLK, D_OUT), lambda i: (i, 0)),
        out_shape=jax.ShapeDtypeStruct((N_NODES, D_OUT), jnp.float32),
    )(feat, feat, deg, deg, W_lin, b_lin.reshape(1, D_OUT), weight)
    return out


# speedup vs baseline: 1.2766x; 1.0049x over previous
import functools

import jax
import jax.numpy as jnp
from jax import lax
from jax.experimental import pallas as pl
from jax.experimental.pallas import tpu as pltpu
from jax.experimental.pallas import tpu_sc as plsc

N_NODES = 10000
N_EDGES = 320000
D_IN = 128
D_DEG = 16
D_OUT = 128

NC = 2
NS = 16
NW = NC * NS
EDGES_PER_WORKER = N_EDGES // NW
CHUNK = 80
N_CHUNKS = EDGES_PER_WORKER // CHUNK
N_ACC = 10240
ROWS_PER_TILE = N_ACC // NS
NROW = 3
NIDX = 4
CPRE = 6000
SPLIT_CHUNK = CPRE // CHUNK
CRE = EDGES_PER_WORKER - CPRE
RELOAD_AT = 50
RELOAD_WAIT_AT = SPLIT_CHUNK - 1

_mesh = plsc.VectorSubcoreMesh(
    core_axis_name="c", subcore_axis_name="s", num_cores=NC, num_subcores=NS
)


@functools.partial(
    pl.kernel,
    out_type=(
        jax.ShapeDtypeStruct((NC * N_ACC, D_IN), jnp.float32),
        jax.ShapeDtypeStruct((NC * N_ACC, D_DEG), jnp.float32),
    ),
    mesh=_mesh,
    scratch_types=[
        pltpu.VMEM((NIDX, CHUNK), jnp.int32),
        pltpu.VMEM((CPRE,), jnp.int32),
        pltpu.VMEM((NROW, CHUNK, D_IN), jnp.float32),
        pltpu.VMEM((CHUNK, D_DEG), jnp.float32),
        pltpu.VMEM_SHARED((N_ACC, D_IN), jnp.float32),
        pltpu.VMEM_SHARED((N_ACC, D_DEG), jnp.float32),
        pltpu.SemaphoreType.DMA,
        pltpu.SemaphoreType.DMA,
        pltpu.SemaphoreType.DMA,
        pltpu.SemaphoreType.DMA,
        pltpu.SemaphoreType.DMA,
    ],
    compiler_params=pltpu.CompilerParams(use_tc_tiling_on_sc=False),
)
def _sc_aggregate(x_hbm, edge_hbm, feat_hbm, deg_hbm, ridx_v, cidx_v, rows_v,
                  ones_v, feat_sh, deg_sh, sem_g, sem_s, sem_d, sem_ir,
                  sem_ic):
    c = lax.axis_index("c")
    s = lax.axis_index("s")
    wid = s * NC + c
    ebase = wid * EDGES_PER_WORKER

    def _load_idx(i, b):
        off = ebase + i * CHUNK
        pltpu.async_copy(edge_hbm.at[0, pl.ds(off, CHUNK)], ridx_v.at[b],
                         sem_ir)

    def _wait_ridx():
        pltpu.make_async_copy(edge_hbm.at[0, pl.ds(0, CHUNK)], ridx_v.at[0],
                              sem_ir).wait()

    def _gather(i, b):
        off = pl.multiple_of(
            jnp.where(i < SPLIT_CHUNK, i * CHUNK, (i - SPLIT_CHUNK) * CHUNK),
            8)
        pltpu.async_copy(x_hbm.at[cidx_v.at[pl.ds(off, CHUNK)]], rows_v.at[b],
                         sem_g)

    def _wait_gather():
        pltpu.make_async_copy(x_hbm.at[cidx_v.at[pl.ds(0, CHUNK)]],
                              rows_v.at[0], sem_g).wait()

    def _wait_scatters():
        pltpu.make_async_copy(
            rows_v.at[0], feat_sh.at[ridx_v.at[0]], sem_s).wait()
        pltpu.make_async_copy(
            ones_v, deg_sh.at[ridx_v.at[0]], sem_d).wait()

    pltpu.async_copy(edge_hbm.at[1, pl.ds(ebase, CPRE)], cidx_v, sem_ic)
    _load_idx(0, 0)
    _load_idx(1, 1)

    def _zero_row(i, carry):
        for j in range(D_IN // 16):
            rows_v[0, i, pl.ds(j * 16, 16)] = jnp.zeros((16,), jnp.float32)
        ones_v[i, :] = jnp.zeros((D_DEG,), jnp.float32)
        return carry

    lax.fori_loop(0, CHUNK, _zero_row, 0)
    row_base = pl.multiple_of(s * ROWS_PER_TILE, 8)
    for k in range(ROWS_PER_TILE // CHUNK):
        pltpu.async_copy(rows_v.at[0],
                         feat_sh.at[pl.ds(row_base + k * CHUNK, CHUNK)], sem_s)
        pltpu.async_copy(ones_v, deg_sh.at[pl.ds(row_base + k * CHUNK, CHUNK)],
                         sem_d)
    for k in range(ROWS_PER_TILE // CHUNK):
        pltpu.make_async_copy(
            rows_v.at[0], feat_sh.at[pl.ds(row_base, CHUNK)], sem_s).wait()
        pltpu.make_async_copy(
            ones_v, deg_sh.at[pl.ds(row_base, CHUNK)], sem_d).wait()

    def _fill_ones(i, carry):
        ones_v[i, :] = jnp.ones((D_DEG,), jnp.float32)
        return carry

    lax.fori_loop(0, CHUNK, _fill_ones, 0)

    pltpu.make_async_copy(edge_hbm.at[1, pl.ds(ebase, CPRE)], cidx_v,
                          sem_ic).wait()
    _gather(0, 0)
    plsc.subcore_barrier()

    def _edge_chunk(i, carry):
        @pl.when(i >= 2)
        def _():
            _wait_scatters()

        @pl.when(i == RELOAD_AT)
        def _():
            pltpu.async_copy(edge_hbm.at[1, pl.ds(ebase + CPRE, CRE)],
                             cidx_v.at[pl.ds(0, CRE)], sem_ic)

        @pl.when(i == RELOAD_WAIT_AT)
        def _():
            pltpu.make_async_copy(edge_hbm.at[1, pl.ds(ebase + CPRE, CRE)],
                                  cidx_v.at[pl.ds(0, CRE)], sem_ic).wait()

        @pl.when(i + 1 < N_CHUNKS)
        def _():
            _gather(i + 1, lax.rem(i + 1, NROW))

        @pl.when(i + 2 < N_CHUNKS)
        def _():
            _load_idx(i + 2, lax.rem(i + 2, NIDX))

        _wait_gather()
        _wait_ridx()
        p = lax.rem(i, NROW)
        b = lax.rem(i, NIDX)
        pltpu.async_copy(rows_v.at[p], feat_sh.at[ridx_v.at[b]], sem_s,
                         add=True)
        pltpu.async_copy(ones_v, deg_sh.at[ridx_v.at[b]], sem_d, add=True)
        return carry

    lax.fori_loop(0, N_CHUNKS, _edge_chunk, 0)
    _wait_scatters()
    _wait_scatters()
    plsc.subcore_barrier()

    out_base = pl.multiple_of(c * N_ACC + row_base, 8)
    pltpu.async_copy(feat_sh.at[pl.ds(row_base, ROWS_PER_TILE)],
                     feat_hbm.at[pl.ds(out_base, ROWS_PER_TILE)], sem_s)
    pltpu.async_copy(deg_sh.at[pl.ds(row_base, ROWS_PER_TILE)],
                     deg_hbm.at[pl.ds(out_base, ROWS_PER_TILE)], sem_d)
    pltpu.make_async_copy(
        feat_sh.at[pl.ds(row_base, ROWS_PER_TILE)],
        feat_hbm.at[pl.ds(out_base, ROWS_PER_TILE)], sem_s).wait()
    pltpu.make_async_copy(
        deg_sh.at[pl.ds(row_base, ROWS_PER_TILE)],
        deg_hbm.at[pl.ds(out_base, ROWS_PER_TILE)], sem_d).wait()


TC_BLK = 5120


def _tc_finalize(fa_ref, fb_ref, da_ref, db_ref, wlin_ref, blin_ref, wsq_ref,
                 out_ref):
    f = fa_ref[...] + fb_ref[...]
    d = da_ref[:, 0:1] + db_ref[:, 0:1]
    norm = f / jnp.maximum(d, 1.0)
    h = lax.dot_general(norm, wlin_ref[...], (((1,), (1,)), ((), ())),
                        preferred_element_type=jnp.float32)
    h = h + blin_ref[...]
    out_ref[...] = jnp.dot(h, wsq_ref[...], preferred_element_type=jnp.float32)


def kernel(x, edge_index, W_lin, b_lin, weight):
    feat, deg = _sc_aggregate(x, edge_index.astype(jnp.int32))
    nblk_half = N_ACC // TC_BLK
    out = pl.pallas_call(
        _tc_finalize,
        grid=((N_NODES + TC_BLK - 1) // TC_BLK,),
        in_specs=[
            pl.BlockSpec((TC_BLK, D_IN), lambda i: (i, 0)),
            pl.BlockSpec((TC_BLK, D_IN), lambda i: (i + nblk_half, 0)),
            pl.BlockSpec((TC_BLK, D_DEG), lambda i: (i, 0)),
            pl.BlockSpec((TC_BLK, D_DEG), lambda i: (i + nblk_half, 0)),
            pl.BlockSpec((D_OUT, D_IN), lambda i: (0, 0)),
            pl.BlockSpec((1, D_OUT), lambda i: (0, 0)),
            pl.BlockSpec((D_OUT, D_OUT), lambda i: (0, 0)),
        ],
        out_specs=pl.BlockSpec((TC_BLK, D_OUT), lambda i: (i, 0)),
        out_shape=jax.ShapeDtypeStruct((N_NODES, D_OUT), jnp.float32),
    )(feat, feat, deg, deg, W_lin, b_lin.reshape(1, D_OUT), weight)
    return out
